# baseline (speedup 1.0000x reference)
import functools

import jax
import jax.numpy as jnp
from jax import lax
from jax.experimental import pallas as pl
from jax.experimental.pallas import tpu as pltpu
from jax.experimental.pallas import tpu_sc as plsc

N_NODES = 10000
D = 128
DH = D // 2
NC, NS = 2, 16
NW = NC * NS
CHUNK = 128
NCHUNK_DEG = 80
NCHUNK_SC = 160
E_PAD = NW * NCHUNK_DEG * CHUNK
EROWS = E_PAD // CHUNK
ROWS_PAD = 10112
SR = ROWS_PAD // NS

_mesh = plsc.VectorSubcoreMesh(
    core_axis_name="c", subcore_axis_name="s", num_cores=NC, num_subcores=NS
)


@functools.partial(
    pl.kernel,
    out_type=jax.ShapeDtypeStruct((NC, ROWS_PAD, 16), jnp.float32),
    mesh=_mesh,
    scratch_types=[
        pltpu.VMEM((NCHUNK_DEG, CHUNK), jnp.int32),
        pltpu.VMEM((CHUNK, 16), jnp.float32),
        pltpu.SemaphoreType.DMA,
        pltpu.VMEM_SHARED((ROWS_PAD, 16), jnp.float32),
    ],
    compiler_params=pltpu.CompilerParams(use_tc_tiling_on_sc=False),
)
def _sc_degree(dst_hbm, ones_hbm, zeros_hbm, cnt_hbm, dst_v, ones_v, dsem, deg_sh):
    cid = lax.axis_index("c")
    sid = lax.axis_index("s")
    wid = sid * NC + cid
    pltpu.sync_copy(zeros_hbm.at[pl.ds(sid * SR, SR)], deg_sh.at[pl.ds(sid * SR, SR)])
    pltpu.sync_copy(dst_hbm.at[pl.ds(wid * NCHUNK_DEG, NCHUNK_DEG)], dst_v)
    pltpu.sync_copy(ones_hbm, ones_v)
    plsc.subcore_barrier()

    def body(k, carry):
        for b in range(8):
            pltpu.async_copy(ones_v, deg_sh.at[dst_v.at[8 * k + b]], dsem, add=True)
        for b in range(8):
            pltpu.make_async_copy(ones_v, deg_sh.at[dst_v.at[8 * k + b]], dsem).wait()
        return carry

    lax.fori_loop(0, NCHUNK_DEG // 8, body, 0)
    plsc.subcore_barrier()
    pltpu.sync_copy(
        deg_sh.at[pl.ds(sid * SR, SR)], cnt_hbm.at[cid, pl.ds(sid * SR, SR)]
    )


NBUF = 4
LA = NBUF // 2
NG = NCHUNK_SC // NBUF


@functools.partial(
    pl.kernel,
    out_type=jax.ShapeDtypeStruct((NC, ROWS_PAD, DH), jnp.float32),
    mesh=_mesh,
    scratch_types=[
        pltpu.VMEM((NCHUNK_SC, CHUNK), jnp.int32),
        pltpu.VMEM((NCHUNK_SC, CHUNK), jnp.int32),
        *[pltpu.VMEM((CHUNK, DH), jnp.float32) for _ in range(NBUF)],
        pltpu.SemaphoreType.DMA((NBUF,)),
        pltpu.SemaphoreType.DMA((NBUF,)),
        pltpu.VMEM_SHARED((ROWS_PAD, DH), jnp.float32),
    ],
    compiler_params=pltpu.CompilerParams(use_tc_tiling_on_sc=False),
)
def _sc_scatter(g_hbm, srcoff_hbm, dst_hbm, zeros_hbm, out_hbm,
                src_v, dst_v, *rest):
    bufs = rest[:NBUF]
    sem_g, sem_s, acc_sh = rest[NBUF:]
    cid = lax.axis_index("c")
    sid = lax.axis_index("s")
    pltpu.sync_copy(zeros_hbm.at[pl.ds(sid * SR, SR)], acc_sh.at[pl.ds(sid * SR, SR)])
    pltpu.sync_copy(
        srcoff_hbm.at[cid, pl.ds(sid * NCHUNK_SC, NCHUNK_SC)], src_v
    )
    pltpu.sync_copy(dst_hbm.at[pl.ds(sid * NCHUNK_SC, NCHUNK_SC)], dst_v)
    plsc.subcore_barrier()

    def start_gather(c, slot):
        pltpu.async_copy(g_hbm.at[src_v.at[c]], bufs[slot], sem_g.at[slot])

    def wait_gather(c, slot):
        pltpu.make_async_copy(g_hbm.at[src_v.at[c]], bufs[slot], sem_g.at[slot]).wait()

    def start_scatter(c, slot):
        pltpu.async_copy(bufs[slot], acc_sh.at[dst_v.at[c]], sem_s.at[slot], add=True)

    def wait_scatter(c, slot):
        pltpu.make_async_copy(bufs[slot], acc_sh.at[dst_v.at[c]], sem_s.at[slot]).wait()

    for b in range(LA):
        start_gather(b, b)
    for b in range(NBUF):
        wait_gather(b, b)
        start_scatter(b, b)
        bn = (b + LA) % NBUF
        if b >= LA:
            wait_scatter(b - LA, bn)
        start_gather(b + LA, bn)

    def group(k, carry):
        j0 = NBUF * k
        for b in range(NBUF):
            j = j0 + b
            wait_gather(j, b)
            start_scatter(j, b)
            bn = (b + LA) % NBUF
            wait_scatter(j - LA, bn)
            start_gather(j + LA, bn)
        return carry

    lax.fori_loop(1, NG - 1, group, 0)

    j0 = NBUF * (NG - 1)
    for b in range(NBUF):
        j = j0 + b
        wait_gather(j, b)
        start_scatter(j, b)
        bn = (b + LA) % NBUF
        wait_scatter(j - LA, bn)
        if b < LA:
            start_gather(j + LA, bn)
    for b in range(LA):
        slot = (j0 + LA + b) % NBUF
        wait_scatter(j0 + LA + b, slot)
    plsc.subcore_barrier()
    pltpu.sync_copy(
        acc_sh.at[pl.ds(sid * SR, SR)], out_hbm.at[cid, pl.ds(sid * SR, SR)]
    )


BR = 1000


def _tc_transform_body(x_ref, w_ref, c0_ref, c1_ref, g_ref, dinv_ref):
    dinv16 = lax.rsqrt(1.0 + c0_ref[...] + c1_ref[...])
    h = jnp.dot(x_ref[...], w_ref[0], preferred_element_type=jnp.float32)
    g_ref[0] = h * dinv16[:, 0:1]
    dinv_ref[...] = dinv16


_tc_transform = pl.pallas_call(
    _tc_transform_body,
    grid=(NC, N_NODES // BR),
    in_specs=[
        pl.BlockSpec((BR, D), lambda c, i: (i, 0)),
        pl.BlockSpec((1, D, DH), lambda c, i: (c, 0, 0)),
        pl.BlockSpec((BR, 16), lambda c, i: (i, 0)),
        pl.BlockSpec((BR, 16), lambda c, i: (i, 0)),
    ],
    out_specs=[
        pl.BlockSpec((1, BR, DH), lambda c, i: (c, i, 0)),
        pl.BlockSpec((BR, 16), lambda c, i: (i, 0)),
    ],
    out_shape=[
        jax.ShapeDtypeStruct((NC, N_NODES, DH), jnp.float32),
        jax.ShapeDtypeStruct((N_NODES, 16), jnp.float32),
    ],
)


def _tc_final_body(a0_ref, a1_ref, g0_ref, g1_ref, dinv_ref, b_ref, out_ref):
    dinv = dinv_ref[:, 0:1]
    left = dinv * (a0_ref[0] + g0_ref[0])
    right = dinv * (a1_ref[0] + g1_ref[0])
    out_ref[...] = jnp.concatenate([left, right], axis=1) + b_ref[...]


_tc_final = pl.pallas_call(
    _tc_final_body,
    grid=(N_NODES // BR,),
    in_specs=[
        pl.BlockSpec((1, BR, DH), lambda i: (0, i, 0)),
        pl.BlockSpec((1, BR, DH), lambda i: (1, i, 0)),
        pl.BlockSpec((1, BR, DH), lambda i: (0, i, 0)),
        pl.BlockSpec((1, BR, DH), lambda i: (1, i, 0)),
        pl.BlockSpec((BR, 16), lambda i: (i, 0)),
        pl.BlockSpec((1, D), lambda i: (0, 0)),
    ],
    out_specs=pl.BlockSpec((BR, D), lambda i: (i, 0)),
    out_shape=jax.ShapeDtypeStruct((N_NODES, D), jnp.float32),
)


def kernel(x, edge_index, node_type, edge_type, W, b):
    del node_type, edge_type
    ei = edge_index.astype(jnp.int32)
    pad = E_PAD - ei.shape[1]
    src = jnp.concatenate([ei[0], jnp.zeros((pad,), jnp.int32)])
    dst = jnp.concatenate([ei[1], jnp.full((pad,), N_NODES, jnp.int32)])
    src2d = src.reshape(EROWS, CHUNK)
    dst2d = dst.reshape(EROWS, CHUNK)
    srcoff = jnp.stack([src2d, src2d + N_NODES])
    ones16 = jnp.ones((CHUNK, 16), jnp.float32)
    zeros16 = jnp.zeros((ROWS_PAD, 16), jnp.float32)
    zeros64 = jnp.zeros((ROWS_PAD, DH), jnp.float32)

    Ws = jnp.stack([W[:, :DH], W[:, DH:]])
    cnt = _sc_degree(dst2d, ones16, zeros16)
    g, dinv16 = _tc_transform(x, Ws, cnt[0], cnt[1])
    g2 = g.reshape(NC * N_NODES, DH)
    acc = _sc_scatter(g2, srcoff, dst2d, zeros64)
    out = _tc_final(acc, acc, g, g, dinv16, b.reshape(1, D))
    return out


# speedup vs baseline: 1.6469x; 1.6469x over previous
import functools

import jax
import jax.numpy as jnp
from jax import lax
from jax.experimental import pallas as pl
from jax.experimental.pallas import tpu as pltpu
from jax.experimental.pallas import tpu_sc as plsc

N_NODES = 10000
D = 128
DH = D // 2
NC, NS = 2, 16
NW = NC * NS
CHUNK = 128
NCHUNK_DEG = 80
NCHUNK_SC = 160
E_PAD = NW * NCHUNK_DEG * CHUNK
EROWS = E_PAD // CHUNK
ROWS_PAD = 10112
SR = ROWS_PAD // NS

_mesh = plsc.VectorSubcoreMesh(
    core_axis_name="c", subcore_axis_name="s", num_cores=NC, num_subcores=NS
)


@functools.partial(
    pl.kernel,
    out_type=jax.ShapeDtypeStruct((NC, ROWS_PAD, 16), jnp.float32),
    mesh=_mesh,
    scratch_types=[
        pltpu.VMEM((NCHUNK_DEG, CHUNK), jnp.int32),
        pltpu.VMEM((CHUNK, 16), jnp.float32),
        pltpu.SemaphoreType.DMA,
        pltpu.VMEM_SHARED((ROWS_PAD, 16), jnp.float32),
    ],
    compiler_params=pltpu.CompilerParams(use_tc_tiling_on_sc=False),
)
def _sc_degree(dst_hbm, ones_hbm, zeros_hbm, cnt_hbm, dst_v, ones_v, dsem, deg_sh):
    cid = lax.axis_index("c")
    sid = lax.axis_index("s")
    wid = sid * NC + cid
    pltpu.sync_copy(zeros_hbm.at[pl.ds(sid * SR, SR)], deg_sh.at[pl.ds(sid * SR, SR)])
    pltpu.sync_copy(dst_hbm.at[pl.ds(wid * NCHUNK_DEG, NCHUNK_DEG)], dst_v)
    pltpu.sync_copy(ones_hbm, ones_v)
    plsc.subcore_barrier()

    def body(k, carry):
        for b in range(8):
            pltpu.async_copy(ones_v, deg_sh.at[dst_v.at[8 * k + b]], dsem, add=True)
        for b in range(8):
            pltpu.make_async_copy(ones_v, deg_sh.at[dst_v.at[8 * k + b]], dsem).wait()
        return carry

    lax.fori_loop(0, NCHUNK_DEG // 8, body, 0)
    plsc.subcore_barrier()
    pltpu.sync_copy(
        deg_sh.at[pl.ds(sid * SR, SR)], cnt_hbm.at[cid, pl.ds(sid * SR, SR)]
    )


NBUF = 4
LA = 2
NG = NCHUNK_SC // NBUF


@functools.partial(
    pl.kernel,
    out_type=jax.ShapeDtypeStruct((NC, ROWS_PAD, DH), jnp.float32),
    mesh=_mesh,
    scratch_types=[
        pltpu.VMEM((NBUF, CHUNK), jnp.int32),
        pltpu.VMEM((NBUF, CHUNK), jnp.int32),
        *[pltpu.VMEM((CHUNK, DH), jnp.float32) for _ in range(NBUF)],
        pltpu.SemaphoreType.DMA((NBUF,)),
        pltpu.SemaphoreType.DMA((NBUF,)),
        pltpu.SemaphoreType.DMA((NBUF,)),
        pltpu.VMEM_SHARED((ROWS_PAD, DH), jnp.float32),
        pltpu.VMEM_SHARED((ROWS_PAD, DH), jnp.float32),
    ],
    compiler_params=pltpu.CompilerParams(use_tc_tiling_on_sc=False),
)
def _sc_scatter(g_hbm, src_hbm, dst_hbm, zeros_hbm, out_hbm,
                srcx, dstx, *rest):
    bufs = rest[:NBUF]
    sem_si, sem_di, sem_s, g_sh, acc_sh = rest[NBUF:]
    cid = lax.axis_index("c")
    sid = lax.axis_index("s")
    base = sid * NCHUNK_SC
    pltpu.sync_copy(zeros_hbm.at[pl.ds(sid * SR, SR)], acc_sh.at[pl.ds(sid * SR, SR)])
    pltpu.sync_copy(g_hbm.at[cid, pl.ds(sid * SR, SR)], g_sh.at[pl.ds(sid * SR, SR)])

    def start_idx(c, slot):
        pltpu.async_copy(src_hbm.at[pl.ds(base + c, 1)],
                         srcx.at[pl.ds(slot, 1)], sem_si.at[slot])
        pltpu.async_copy(dst_hbm.at[pl.ds(base + c, 1)],
                         dstx.at[pl.ds(slot, 1)], sem_di.at[slot])

    def wait_idx(c, slot):
        pltpu.make_async_copy(src_hbm.at[pl.ds(base + c, 1)],
                              srcx.at[pl.ds(slot, 1)], sem_si.at[slot]).wait()
        pltpu.make_async_copy(dst_hbm.at[pl.ds(base + c, 1)],
                              dstx.at[pl.ds(slot, 1)], sem_di.at[slot]).wait()

    def gather(slot):
        pltpu.sync_copy(g_sh.at[srcx.at[slot]], bufs[slot])

    def start_scatter(slot):
        pltpu.async_copy(bufs[slot], acc_sh.at[dstx.at[slot]],
                         sem_s.at[slot], add=True)

    def wait_scatter(slot):
        pltpu.make_async_copy(bufs[slot], acc_sh.at[dstx.at[slot]],
                              sem_s.at[slot]).wait()

    plsc.subcore_barrier()
    start_idx(0, 0)
    start_idx(1, 1)
    for b in range(NBUF):
        wait_idx(b, b)
        gather(b)
        start_scatter(b)
        b2 = (b + LA) % NBUF
        if b >= LA:
            wait_scatter(b2)
        start_idx(b + LA, b2)

    def group(k, carry):
        j0 = NBUF * k
        for b in range(NBUF):
            j = j0 + b
            wait_idx(j, b)
            gather(b)
            start_scatter(b)
            b2 = (b + LA) % NBUF
            wait_scatter(b2)
            start_idx(j + LA, b2)
        return carry

    lax.fori_loop(1, NG - 1, group, 0)

    j0 = NBUF * (NG - 1)
    for b in range(NBUF):
        j = j0 + b
        wait_idx(j, b)
        gather(b)
        start_scatter(b)
        b2 = (b + LA) % NBUF
        wait_scatter(b2)
        if b < LA:
            start_idx(j + LA, b2)
    for b in range(LA):
        wait_scatter(LA + b)
    plsc.subcore_barrier()
    pltpu.sync_copy(
        acc_sh.at[pl.ds(sid * SR, SR)], out_hbm.at[cid, pl.ds(sid * SR, SR)]
    )


BR = 1000


def _tc_transform_body(x_ref, w_ref, c0_ref, c1_ref, g_ref, dinv_ref):
    dinv16 = lax.rsqrt(1.0 + c0_ref[...] + c1_ref[...])
    h = jnp.dot(x_ref[...], w_ref[0], preferred_element_type=jnp.float32)
    g_ref[0] = h * dinv16[:, 0:1]
    dinv_ref[...] = dinv16


_tc_transform = pl.pallas_call(
    _tc_transform_body,
    grid=(NC, N_NODES // BR),
    in_specs=[
        pl.BlockSpec((BR, D), lambda c, i: (i, 0)),
        pl.BlockSpec((1, D, DH), lambda c, i: (c, 0, 0)),
        pl.BlockSpec((BR, 16), lambda c, i: (i, 0)),
        pl.BlockSpec((BR, 16), lambda c, i: (i, 0)),
    ],
    out_specs=[
        pl.BlockSpec((1, BR, DH), lambda c, i: (c, i, 0)),
        pl.BlockSpec((BR, 16), lambda c, i: (i, 0)),
    ],
    out_shape=[
        jax.ShapeDtypeStruct((NC, ROWS_PAD, DH), jnp.float32),
        jax.ShapeDtypeStruct((N_NODES, 16), jnp.float32),
    ],
)


def _tc_final_body(a0_ref, a1_ref, g0_ref, g1_ref, dinv_ref, b_ref, out_ref):
    dinv = dinv_ref[:, 0:1]
    left = dinv * (a0_ref[0] + g0_ref[0])
    right = dinv * (a1_ref[0] + g1_ref[0])
    out_ref[...] = jnp.concatenate([left, right], axis=1) + b_ref[...]


_tc_final = pl.pallas_call(
    _tc_final_body,
    grid=(N_NODES // BR,),
    in_specs=[
        pl.BlockSpec((1, BR, DH), lambda i: (0, i, 0)),
        pl.BlockSpec((1, BR, DH), lambda i: (1, i, 0)),
        pl.BlockSpec((1, BR, DH), lambda i: (0, i, 0)),
        pl.BlockSpec((1, BR, DH), lambda i: (1, i, 0)),
        pl.BlockSpec((BR, 16), lambda i: (i, 0)),
        pl.BlockSpec((1, D), lambda i: (0, 0)),
    ],
    out_specs=pl.BlockSpec((BR, D), lambda i: (i, 0)),
    out_shape=jax.ShapeDtypeStruct((N_NODES, D), jnp.float32),
)


def kernel(x, edge_index, node_type, edge_type, W, b):
    del node_type, edge_type
    ei = edge_index.astype(jnp.int32)
    pad = E_PAD - ei.shape[1]
    src = jnp.concatenate([ei[0], jnp.zeros((pad,), jnp.int32)])
    dst = jnp.concatenate([ei[1], jnp.full((pad,), N_NODES, jnp.int32)])
    src2d = src.reshape(EROWS, CHUNK)
    dst2d = dst.reshape(EROWS, CHUNK)
    ones16 = jnp.ones((CHUNK, 16), jnp.float32)
    zeros16 = jnp.zeros((ROWS_PAD, 16), jnp.float32)
    zeros64 = jnp.zeros((ROWS_PAD, DH), jnp.float32)

    Ws = jnp.stack([W[:, :DH], W[:, DH:]])
    cnt = _sc_degree(dst2d, ones16, zeros16)
    g, dinv16 = _tc_transform(x, Ws, cnt[0], cnt[1])
    acc = _sc_scatter(g, src2d, dst2d, zeros64)
    out = _tc_final(acc, acc, g, g, dinv16, b.reshape(1, D))
    return out
